# Initial kernel scaffold; baseline (speedup 1.0000x reference)
import jax
import jax.numpy as jnp
from jax.experimental import pallas as pl


def kernel(x, weight, bias):
    raise NotImplementedError("write your pallas kernel here")


# speedup vs baseline: 2.6640x; 2.6640x over previous
import functools

import jax
import jax.numpy as jnp
from jax.experimental import pallas as pl
from jax.experimental.pallas import tpu as pltpu

_EPS = 1e-05
_B = 512
_NIT = 5
_LN_BM = 512
_MM_BM = 256
_VMEM = 58 * 1024 * 1024


def _ln_stats_kernel(nblk, steps, x_ref, xn_ref, gram_ref, csum_ref):
    s = pl.program_id(1)
    xv = x_ref[...]
    d = xv.shape[-1]
    mean = jnp.mean(xv, axis=-1, keepdims=True)
    xc = xv - mean
    var = jnp.sum(xc * xc, axis=-1, keepdims=True) * (1.0 / (d - 1))
    xn = xc / (jnp.sqrt(var) + _EPS)
    xn_ref[...] = xn.astype(jnp.bfloat16)

    @pl.when(s == 0)
    def _():
        gram_ref[...] = jnp.zeros_like(gram_ref)
        csum_ref[...] = jnp.zeros_like(csum_ref)

    g = gram_ref[0]
    for j in range(nblk):
        xj = xn[:, j * _B:(j + 1) * _B]
        g = g + jax.lax.dot_general(
            xj, xj, (((0,), (0,)), ((), ())),
            preferred_element_type=jnp.float32)
    gram_ref[0] = g
    csum_ref[0, 0:1, :] += jnp.sum(xn, axis=0, keepdims=True)


def _ns_kernel(n_rows, gram_ref, s_ref, c_ref):
    g = gram_ref[...]
    s = s_ref[...]
    dim = g.shape[0]
    inv_n = 1.0 / n_rows
    outer = jax.lax.dot_general(
        s, s, (((0,), (0,)), ((), ())),
        preferred_element_type=jnp.float32)
    row = jax.lax.broadcasted_iota(jnp.int32, (dim, dim), 0)
    col = jax.lax.broadcasted_iota(jnp.int32, (dim, dim), 1)
    eye = jnp.where(row == col, jnp.float32(1.0), jnp.float32(0.0))
    cov = g * inv_n - outer * (inv_n * inv_n) + _EPS * eye
    norm_a = jnp.sqrt(jnp.sum(cov * cov))
    y = cov * (1.0 / norm_a)
    z = eye
    for _ in range(_NIT):
        t = 1.5 * eye - 0.5 * jnp.dot(z, y, preferred_element_type=jnp.float32)
        y = jnp.dot(y, t, preferred_element_type=jnp.float32)
        z = jnp.dot(t, z, preferred_element_type=jnp.float32)
    c_ref[...] = z * jax.lax.rsqrt(norm_a)


def _w_xform_kernel(nblk, w_ref, c_ref, wt_ref):
    c = c_ref[...]
    for j in range(nblk):
        wj = w_ref[:, j * _B:(j + 1) * _B]
        r = jax.lax.dot_general(
            c, wj, (((0,), (1,)), ((), ())),
            preferred_element_type=jnp.float32)
        wt_ref[j * _B:(j + 1) * _B, :] = r.astype(jnp.bfloat16)


def _final_mm_kernel(x_ref, w_ref, m_ref, b_ref, o_ref):
    xc = x_ref[...] - m_ref[...]
    o_ref[...] = jnp.dot(
        xc, w_ref[...], preferred_element_type=jnp.float32) + b_ref[...]


@jax.jit
def kernel(x, weight, bias):
    n_rows_x, d = x.shape
    d_out = weight.shape[0]
    nblk = d // _B
    n = (n_rows_x * d) // _B
    steps = n_rows_x // _LN_BM

    xn, gram_p, csum_p = pl.pallas_call(
        functools.partial(_ln_stats_kernel, nblk, steps // 2),
        grid=(2, steps // 2),
        in_specs=[pl.BlockSpec((_LN_BM, d), lambda c, s: (c * (steps // 2) + s, 0))],
        out_specs=[
            pl.BlockSpec((_LN_BM, d), lambda c, s: (c * (steps // 2) + s, 0)),
            pl.BlockSpec((1, _B, _B), lambda c, s: (c, 0, 0)),
            pl.BlockSpec((1, 8, d), lambda c, s: (c, 0, 0)),
        ],
        out_shape=[
            jax.ShapeDtypeStruct((n_rows_x, d), jnp.bfloat16),
            jax.ShapeDtypeStruct((2, _B, _B), jnp.float32),
            jax.ShapeDtypeStruct((2, 8, d), jnp.float32),
        ],
        compiler_params=pltpu.CompilerParams(
            dimension_semantics=("parallel", "arbitrary"),
            vmem_limit_bytes=_VMEM,
        ),
        name="ln_stats",
    )(x)

    gram = gram_p[0] + gram_p[1]
    svec = csum_p.sum(axis=(0, 1)).reshape(nblk, _B).sum(axis=0)

    c_mat = pl.pallas_call(
        functools.partial(_ns_kernel, float(n)),
        out_shape=jax.ShapeDtypeStruct((_B, _B), jnp.float32),
        compiler_params=pltpu.CompilerParams(vmem_limit_bytes=_VMEM),
        name="ns_isqrt",
    )(gram, svec.reshape(1, _B))

    wt = pl.pallas_call(
        functools.partial(_w_xform_kernel, nblk),
        grid=(nblk,),
        in_specs=[
            pl.BlockSpec((d_out // nblk, d), lambda o: (o, 0)),
            pl.BlockSpec((_B, _B), lambda o: (0, 0)),
        ],
        out_specs=pl.BlockSpec((d, d_out // nblk), lambda o: (0, o)),
        out_shape=jax.ShapeDtypeStruct((d, d_out), jnp.bfloat16),
        compiler_params=pltpu.CompilerParams(
            dimension_semantics=("parallel",),
            vmem_limit_bytes=_VMEM,
        ),
        name="w_xform",
    )(weight, c_mat)

    mean_full = jnp.tile(svec * (1.0 / n), nblk).reshape(1, d).astype(jnp.bfloat16)
    bias_row = bias.reshape(1, d_out)

    out = pl.pallas_call(
        _final_mm_kernel,
        grid=(n_rows_x // _MM_BM,),
        in_specs=[
            pl.BlockSpec((_MM_BM, d), lambda i: (i, 0)),
            pl.BlockSpec((d, d_out), lambda i: (0, 0)),
            pl.BlockSpec((1, d), lambda i: (0, 0)),
            pl.BlockSpec((1, d_out), lambda i: (0, 0)),
        ],
        out_specs=pl.BlockSpec((_MM_BM, d_out), lambda i: (i, 0)),
        out_shape=jax.ShapeDtypeStruct((n_rows_x, d_out), jnp.float32),
        compiler_params=pltpu.CompilerParams(
            dimension_semantics=("parallel",),
            vmem_limit_bytes=_VMEM,
        ),
        name="final_mm",
    )(xn, wt, mean_full, bias_row)
    return out
